# Initial kernel scaffold; baseline (speedup 1.0000x reference)
import functools

import numpy as np
import jax
import jax.numpy as jnp
from jax import lax
from jax.experimental import pallas as pl
from jax.experimental.pallas import tpu as pltpu
from jax.experimental.pallas import tpu_sc as plsc

_CUTOFF = 5.0
_LOG2 = float(np.log(2.0))
_PI = float(np.pi)


def _ssp(v):
    return jnp.maximum(v, 0.0) + jnp.log1p(jnp.exp(-jnp.abs(v))) - _LOG2


def _filter_body(f_ref, r_ref, m_ref, w1_ref, b1_ref, w2_ref, b2_ref, wf_ref):
    h = jnp.dot(f_ref[...], w1_ref[...], preferred_element_type=jnp.float32)
    h = _ssp(h + b1_ref[...])
    w = jnp.dot(h, w2_ref[...], preferred_element_type=jnp.float32) + b2_ref[...]
    r = r_ref[...]
    c = 0.5 * (jnp.cos(r * (_PI / _CUTOFF)) + 1.0)
    c = jnp.where(r < _CUTOFF, c, 0.0) * m_ref[...]
    wf_ref[...] = w * c


def _in2f_body(x_ref, w_ref, y_ref):
    y_ref[...] = jnp.dot(x_ref[...], w_ref[...], preferred_element_type=jnp.float32)


def _out_body(a_ref, w_ref, b_ref, o_ref):
    o_ref[...] = _ssp(
        jnp.dot(a_ref[...], w_ref[...], preferred_element_type=jnp.float32)
        + b_ref[...]
    )


@functools.lru_cache(maxsize=None)
def _make_sc_agg(pairs, nbh, feat, ch):
    info = plsc.get_sparse_core_info()
    nc, ns, lanes = info.num_cores, info.num_subcores, info.num_lanes
    nw = nc * ns
    ppw = pairs // nw
    nit = ppw // ch
    nf = feat // lanes
    mesh = plsc.VectorSubcoreMesh(core_axis_name="c", subcore_axis_name="s")

    @functools.partial(
        pl.kernel,
        mesh=mesh,
        out_type=jax.ShapeDtypeStruct((pairs, feat), jnp.float32),
        scratch_types=[
            pltpu.VMEM((ch, nbh), jnp.int32),
            pltpu.VMEM((ch, nbh, feat), jnp.float32),
            pltpu.VMEM((ch, nbh, feat), jnp.float32),
            pltpu.VMEM((ch, feat), jnp.float32),
            pltpu.SemaphoreType.DMA,
        ],
    )
    def agg(y_hbm, wf_hbm, idx_hbm, out_hbm, idx_v, rows_v, wf_v, acc_v, sem):
        wid = lax.axis_index("s") * nc + lax.axis_index("c")
        base = wid * ppw

        def chunk(ci, carry):
            p0 = base + ci * ch
            pltpu.sync_copy(idx_hbm.at[pl.ds(p0, ch)], idx_v)
            cps = [
                pltpu.async_copy(y_hbm.at[idx_v.at[c]], rows_v.at[c], sem)
                for c in range(ch)
            ]
            pltpu.sync_copy(wf_hbm.at[pl.ds(p0, ch)], wf_v)
            for cp in cps:
                cp.wait()
            for c in range(ch):
                def kstep(k, accs, c=c):
                    return tuple(
                        accs[j]
                        + rows_v[c, k, pl.ds(j * lanes, lanes)]
                        * wf_v[c, k, pl.ds(j * lanes, lanes)]
                        for j in range(nf)
                    )
                accs = lax.fori_loop(
                    0, nbh, kstep,
                    tuple(jnp.zeros((lanes,), jnp.float32) for _ in range(nf)),
                )
                for j in range(nf):
                    acc_v[c, pl.ds(j * lanes, lanes)] = accs[j]
            pltpu.sync_copy(acc_v, out_hbm.at[pl.ds(p0, ch)])
            return carry

        lax.fori_loop(0, nit, chunk, 0)

    return agg


def kernel(x, r_ij, neighbors, pairwise_mask, f_ij, W1, b1, W2, b2, Win, Wout, bout):
    B, N, F = x.shape
    NBH = neighbors.shape[2]
    NG = f_ij.shape[3]
    ROWS = B * N * NBH
    PAIRS = B * N

    f2 = f_ij.reshape(ROWS, NG)
    r2 = r_ij.reshape(ROWS, 1)
    m2 = pairwise_mask.reshape(ROWS, 1)

    RB = 2048
    wf = pl.pallas_call(
        _filter_body,
        grid=(ROWS // RB,),
        in_specs=[
            pl.BlockSpec((RB, NG), lambda i: (i, 0)),
            pl.BlockSpec((RB, 1), lambda i: (i, 0)),
            pl.BlockSpec((RB, 1), lambda i: (i, 0)),
            pl.BlockSpec((NG, F), lambda i: (0, 0)),
            pl.BlockSpec((1, F), lambda i: (0, 0)),
            pl.BlockSpec((F, F), lambda i: (0, 0)),
            pl.BlockSpec((1, F), lambda i: (0, 0)),
        ],
        out_specs=pl.BlockSpec((RB, F), lambda i: (i, 0)),
        out_shape=jax.ShapeDtypeStruct((ROWS, F), jnp.float32),
    )(f2, r2, m2, W1, b1.reshape(1, F), W2, b2.reshape(1, F))

    y2 = pl.pallas_call(
        _in2f_body,
        out_shape=jax.ShapeDtypeStruct((PAIRS, F), jnp.float32),
    )(x.reshape(PAIRS, F), Win)

    nb32 = neighbors.astype(jnp.int32)
    gidx = (nb32 + (jnp.arange(B, dtype=jnp.int32) * N)[:, None, None]).reshape(
        PAIRS, NBH
    )
    agg = _make_sc_agg(PAIRS, NBH, F, 4)(y2, wf.reshape(PAIRS, NBH, F), gidx)

    out = pl.pallas_call(
        _out_body,
        out_shape=jax.ShapeDtypeStruct((PAIRS, F), jnp.float32),
    )(agg, Wout, bout.reshape(1, F))
    return out.reshape(B, N, F)


# speedup vs baseline: 6.9359x; 6.9359x over previous
import functools

import numpy as np
import jax
import jax.numpy as jnp
from jax import lax
from jax.experimental import pallas as pl
from jax.experimental.pallas import tpu as pltpu
from jax.experimental.pallas import tpu_sc as plsc

_CUTOFF = 5.0
_LOG2 = float(np.log(2.0))
_PI = float(np.pi)


def _ssp(v):
    return jnp.maximum(v, 0.0) + jnp.log1p(jnp.exp(-jnp.abs(v))) - _LOG2


def _cutoff_body(r_ref, m_ref, c_ref):
    r = r_ref[...]
    c = 0.5 * (jnp.cos(r * (_PI / _CUTOFF)) + 1.0)
    c_ref[...] = jnp.where(r < _CUTOFF, c, 0.0) * m_ref[...]


def _filter_body(f_ref, c_ref, w1_ref, b1_ref, w2_ref, b2_ref, wf_ref):
    h = jnp.dot(f_ref[...], w1_ref[...], preferred_element_type=jnp.float32)
    h = _ssp(h + b1_ref[...])
    w = jnp.dot(h, w2_ref[...], preferred_element_type=jnp.float32) + b2_ref[...]
    wf_ref[...] = w * c_ref[...]


def _in2f_body(x_ref, w_ref, y_ref):
    y_ref[...] = jnp.dot(x_ref[...], w_ref[...], preferred_element_type=jnp.float32)


def _out_body(a_ref, w_ref, b_ref, o_ref):
    o_ref[...] = _ssp(
        jnp.dot(a_ref[...], w_ref[...], preferred_element_type=jnp.float32)
        + b_ref[...]
    )


@functools.lru_cache(maxsize=None)
def _make_sc_agg(pairs, nbh, feat, ch):
    info = plsc.get_sparse_core_info()
    nc, ns, lanes = info.num_cores, info.num_subcores, info.num_lanes
    nw = nc * ns
    ppw = pairs // nw
    nit = ppw // ch
    nf = feat // lanes
    mesh = plsc.VectorSubcoreMesh(core_axis_name="c", subcore_axis_name="s")

    @functools.partial(
        pl.kernel,
        mesh=mesh,
        out_type=jax.ShapeDtypeStruct((pairs, feat), jnp.float32),
        scratch_types=[
            pltpu.VMEM((ch, nbh), jnp.int32),
            pltpu.VMEM((ch, nbh, feat), jnp.float32),
            pltpu.VMEM((ch, nbh, feat), jnp.float32),
            pltpu.VMEM((ch, feat), jnp.float32),
            pltpu.SemaphoreType.DMA,
        ],
    )
    def agg(y_hbm, wf_hbm, idx_hbm, out_hbm, idx_v, rows_v, wf_v, acc_v, sem):
        wid = lax.axis_index("s") * nc + lax.axis_index("c")
        base = wid * ppw

        def chunk(ci, carry):
            p0 = base + ci * ch
            pltpu.sync_copy(idx_hbm.at[pl.ds(p0, ch)], idx_v)
            cps = [
                pltpu.async_copy(y_hbm.at[idx_v.at[c]], rows_v.at[c], sem)
                for c in range(ch)
            ]
            pltpu.sync_copy(wf_hbm.at[pl.ds(p0, ch)], wf_v)
            for cp in cps:
                cp.wait()
            for c in range(ch):
                def kstep(k, accs, c=c):
                    return tuple(
                        accs[j]
                        + rows_v[c, k, pl.ds(j * lanes, lanes)]
                        * wf_v[c, k, pl.ds(j * lanes, lanes)]
                        for j in range(nf)
                    )
                accs = lax.fori_loop(
                    0, nbh, kstep,
                    tuple(jnp.zeros((lanes,), jnp.float32) for _ in range(nf)),
                )
                for j in range(nf):
                    acc_v[c, pl.ds(j * lanes, lanes)] = accs[j]
            pltpu.sync_copy(acc_v, out_hbm.at[pl.ds(p0, ch)])
            return carry

        lax.fori_loop(0, nit, chunk, 0)

    return agg


def kernel(x, r_ij, neighbors, pairwise_mask, f_ij, W1, b1, W2, b2, Win, Wout, bout):
    B, N, F = x.shape
    NBH = neighbors.shape[2]
    NG = f_ij.shape[3]
    ROWS = B * N * NBH
    PAIRS = B * N

    f2 = f_ij.reshape(ROWS, NG)
    rd = r_ij.reshape(ROWS // 128, 128)
    md = pairwise_mask.reshape(ROWS // 128, 128)

    CB = 256
    c2 = pl.pallas_call(
        _cutoff_body,
        grid=(ROWS // 128 // CB,),
        in_specs=[
            pl.BlockSpec((CB, 128), lambda i: (i, 0)),
            pl.BlockSpec((CB, 128), lambda i: (i, 0)),
        ],
        out_specs=pl.BlockSpec((CB, 128), lambda i: (i, 0)),
        out_shape=jax.ShapeDtypeStruct((ROWS // 128, 128), jnp.float32),
    )(rd, md).reshape(ROWS, 1)

    RB = 2048
    wf = pl.pallas_call(
        _filter_body,
        grid=(ROWS // RB,),
        in_specs=[
            pl.BlockSpec((RB, NG), lambda i: (i, 0)),
            pl.BlockSpec((RB, 1), lambda i: (i, 0)),
            pl.BlockSpec((NG, F), lambda i: (0, 0)),
            pl.BlockSpec((1, F), lambda i: (0, 0)),
            pl.BlockSpec((F, F), lambda i: (0, 0)),
            pl.BlockSpec((1, F), lambda i: (0, 0)),
        ],
        out_specs=pl.BlockSpec((RB, F), lambda i: (i, 0)),
        out_shape=jax.ShapeDtypeStruct((ROWS, F), jnp.float32),
    )(f2, c2, W1, b1.reshape(1, F), W2, b2.reshape(1, F))

    y2 = pl.pallas_call(
        _in2f_body,
        out_shape=jax.ShapeDtypeStruct((PAIRS, F), jnp.float32),
    )(x.reshape(PAIRS, F), Win)

    nb32 = neighbors.astype(jnp.int32)
    gidx = (nb32 + (jnp.arange(B, dtype=jnp.int32) * N)[:, None, None]).reshape(
        PAIRS, NBH
    )
    agg = _make_sc_agg(PAIRS, NBH, F, 4)(y2, wf.reshape(PAIRS, NBH, F), gidx)

    out = pl.pallas_call(
        _out_body,
        out_shape=jax.ShapeDtypeStruct((PAIRS, F), jnp.float32),
    )(agg, Wout, bout.reshape(1, F))
    return out.reshape(B, N, F)
